# baseline (speedup 1.0000x reference)
import functools

import jax
import jax.numpy as jnp
from jax import lax
from jax.experimental import pallas as pl
from jax.experimental.pallas import tpu as pltpu
from jax.experimental.pallas import tpu_sc as plsc

NV_PREV = 10242
NV = 40962
NF = 81920
CIN = 64
COUT = 64

NC, NS = 2, 16
NW = NC * NS
NVPAD = 43008


def _split_sizes(n, parts):
  base = (n // parts) // 8 * 8
  sizes = [base] * (parts - 1)
  sizes.append(n - base * (parts - 1))
  assert all(s > 0 and s % 8 == 0 for s in sizes)
  return sizes


def _make_sc_spmm(n_rows, K, SW, wsets, R, nseg, seg_stride, fold, name,
                  skew=0):
  OW = wsets * SW
  cpw = n_rows // (NW * R)
  Kn = K // nseg
  seg_len = R * Kn
  mesh = plsc.VectorSubcoreMesh(core_axis_name="c", subcore_axis_name="s",
                                num_cores=NC, num_subcores=NS)
  gsizes = _split_sizes(R * K, 4)
  goffs = [sum(gsizes[:i]) for i in range(4)]

  if fold:
    wscratch = [pltpu.VMEM((R * K,), jnp.float32),
                pltpu.VMEM((R * K,), jnp.float32),
                pltpu.VMEM((R, 3), jnp.float32),
                pltpu.VMEM((R, 3), jnp.float32),
                pltpu.VMEM((R, 3), jnp.float32),
                pltpu.VMEM((R, 3), jnp.float32)]
  else:
    wscratch = [pltpu.VMEM((R * K,), jnp.float32),
                pltpu.VMEM((R * K,), jnp.float32)]

  @functools.partial(
      pl.kernel,
      out_type=jax.ShapeDtypeStruct((n_rows, OW), jnp.float32),
      mesh=mesh,
      scratch_types=[
          pltpu.VMEM((R * K,), jnp.int32),
          pltpu.VMEM((R * K,), jnp.int32),
          *wscratch,
          pltpu.VMEM((R * K, SW), jnp.float32),
          pltpu.VMEM((R * K, SW), jnp.float32),
          pltpu.VMEM((R, OW), jnp.float32),
          pltpu.VMEM((R, OW), jnp.float32),
          pltpu.SemaphoreType.DMA,
          pltpu.SemaphoreType.DMA,
          pltpu.SemaphoreType.DMA,
          pltpu.SemaphoreType.DMA,
      ],
      compiler_params=pltpu.CompilerParams(needs_layout_passes=False,
                                           use_tc_tiling_on_sc=False),
      name=name,
  )
  def spmm(src_hbm, cols_hbm, *rest):
    if fold:
      gv_hbm, ew_hbm, ns_hbm, out_hbm = rest[0], rest[1], rest[2], rest[3]
      (idx0, idx1, gv0, gv1, ew0, ew1, ns0, ns1, r0, r1, o0, o1,
       si0, si1, sr0, sr1) = rest[4:]
      w_b = ((gv0, ew0, ns0), (gv1, ew1, ns1))
    else:
      w_hbm, out_hbm = rest[0], rest[1]
      (idx0, idx1, w0, w1, r0, r1, o0, o1,
       si0, si1, sr0, sr1) = rest[2:]
      w_b = ((w0,), (w1,))
    idx_b, rows_b, out_b = (idx0, idx1), (r0, r1), (o0, o1)
    si, sr = (si0, si1), (sr0, sr1)
    cid = lax.axis_index("c")
    sid = lax.axis_index("s")
    if skew:
      cpw_a, cpw_b = cpw - skew, cpw + skew
      my_cpw = jnp.where(cid == 0, cpw_a, cpw_b)
      g0 = jnp.where(cid == 0, sid * cpw_a, NS * cpw_a + sid * cpw_b)
    else:
      my_cpw = cpw
      g0 = (sid * NC + cid) * cpw
    lane = lax.iota(jnp.int32, 16)
    cols_s = [lane + gi * 16 for gi in range(SW // 16)]

    def idx_copies(g, ib, wb, sem):
      ds = []
      for d in range(nseg):
        sl = pl.ds(d * seg_stride + g * seg_len, seg_len)
        ds.append(pltpu.make_async_copy(
            cols_hbm.at[sl], ib.at[pl.ds(d * seg_len, seg_len)], sem))
      if fold:
        gv_v, ew_v, ns_v = wb
        for d in range(nseg):
          sl = pl.ds(d * seg_stride + g * seg_len, seg_len)
          ds.append(pltpu.make_async_copy(
              gv_hbm.at[sl], gv_v.at[pl.ds(d * seg_len, seg_len)], sem))
        ds.append(pltpu.make_async_copy(
            ew_hbm.at[pl.ds(g * R, R)], ew_v, sem))
        ds.append(pltpu.make_async_copy(
            ns_hbm.at[pl.ds(g * R, R)], ns_v, sem))
      else:
        for d in range(nseg):
          sl = pl.ds(d * seg_stride + g * seg_len, seg_len)
          ds.append(pltpu.make_async_copy(
              w_hbm.at[sl], wb[0].at[pl.ds(d * seg_len, seg_len)], sem))
      return ds

    def gather_copies(ib, rb, sem):
      return [
          pltpu.make_async_copy(src_hbm.at[ib.at[pl.ds(o, s)]],
                                rb.at[pl.ds(o, s)], sem)
          for o, s in zip(goffs, gsizes)
      ]

    def compute(rows_v, wb, out_v):
      def f_body(f, carry):
        fvec = jnp.broadcast_to(f * Kn, (16,))
        ovec = jnp.broadcast_to(f, (16,))
        if fold:
          gv_v, ew_v, ns_v = wb
          gw = [plsc.load_gather(gv_v, [fvec + (d * seg_len + j)])
                for d in range(nseg) for j in range(Kn)]
          eww = [plsc.load_gather(ew_v, [ovec, jnp.full((16,), d, jnp.int32)])
                 for d in range(nseg)]
          nsw = [plsc.load_gather(ns_v, [ovec, jnp.full((16,), d, jnp.int32)])
                 for d in range(nseg)]
          wv = [[eww[k // Kn] * gw[k] for k in range(K)],
                [nsw[k // Kn] * gw[k] for k in range(K)]]
        else:
          wv = [[plsc.load_gather(wb[0], [fvec + (d * seg_len + j)])
                 for d in range(nseg) for j in range(Kn)]]
        rowbase = [fvec + (d * seg_len + j)
                   for d in range(nseg) for j in range(Kn)]
        for gi in range(SW // 16):
          accs = [jnp.zeros((16,), jnp.float32) for _ in range(wsets)]
          for k in range(K):
            v = plsc.load_gather(rows_v, [rowbase[k], cols_s[gi]])
            for ws in range(wsets):
              accs[ws] = accs[ws] + wv[ws][k] * v
          for ws in range(wsets):
            plsc.store_scatter(out_v, [ovec, cols_s[gi] + ws * SW], accs[ws])
        return carry

      lax.fori_loop(0, R, f_body, 0)

    for d in idx_copies(g0, idx0, w_b[0], si0):
      d.start()
    for d in idx_copies(g0, idx0, w_b[0], si0):
      d.wait()
    for d in gather_copies(idx0, r0, sr0):
      d.start()
    for d in idx_copies(g0 + 1, idx1, w_b[1], si1):
      d.start()

    def pair_body(i, carry):
      for p in range(2):
        ch = 2 * i + p
        g = g0 + ch
        q = 1 - p

        @pl.when(ch + 1 < my_cpw)
        def _fire_gather():
          for d in idx_copies(g + 1, idx_b[q], w_b[q], si[q]):
            d.wait()
          for d in gather_copies(idx_b[q], rows_b[q], sr[q]):
            d.start()

        @pl.when(ch < my_cpw)
        def _main():
          for d in gather_copies(idx_b[p], rows_b[p], sr[p]):
            d.wait()
          compute(rows_b[p], w_b[p], out_b[p])
          pltpu.sync_copy(out_b[p], out_hbm.at[pl.ds(g * R, R)])

        @pl.when(ch + 2 < my_cpw)
        def _prefetch_idx():
          for d in idx_copies(g + 2, idx_b[p], w_b[p], si[p]):
            d.start()

      return carry

    lax.fori_loop(0, (cpw + abs(skew)) // 2, pair_body, 0)

  return spmm


_NBLK = 512
_NGRID = (NV + _NBLK - 1) // _NBLK


def _mix_kernel(inp_ref, lap_ref, gv_ref, at_ref, bias_ref, out_ref):
  feat = jnp.concatenate([inp_ref[...], lap_ref[...], gv_ref[...]], axis=1)
  out = lax.dot_general(at_ref[...], feat, (((1,), (1,)), ((), ())),
                        preferred_element_type=jnp.float32)
  out_ref[...] = out + bias_ref[...]


@functools.lru_cache(maxsize=None)
def _make_mix():
  return pl.pallas_call(
      _mix_kernel,
      grid=(_NGRID,),
      in_specs=[
          pl.BlockSpec((_NBLK, CIN), lambda i: (i, 0)),
          pl.BlockSpec((_NBLK, CIN), lambda i: (i, 0)),
          pl.BlockSpec((_NBLK, 2 * CIN), lambda i: (i, 0)),
          pl.BlockSpec((COUT, 4 * CIN), lambda i: (0, 0)),
          pl.BlockSpec((COUT, 1), lambda i: (0, 0)),
      ],
      out_specs=pl.BlockSpec((COUT, _NBLK), lambda i: (0, i)),
      out_shape=jax.ShapeDtypeStruct((COUT, NV), jnp.float32),
  )


def kernel(x, verts, G_rows, G_cols, G_vals, NS_w, EW, L_rows, L_cols, L_vals,
           F2V_rows, F2V_cols, F2V_vals, coeffs, bias):
  f32 = jnp.float32
  i32 = jnp.int32
  inp_t = jnp.concatenate(
      [x[0].T, jnp.ones((NVPAD - NV_PREV, CIN), f32)], axis=0)

  npad = NVPAD - NV
  cols_l = jnp.concatenate([L_cols, jnp.zeros((npad * 7,), i32)])
  w_l = jnp.concatenate([L_vals, jnp.zeros((npad * 7,), f32)])
  cols_v = jnp.concatenate([F2V_cols, jnp.zeros((npad * 6,), i32)])
  w_v = jnp.concatenate([F2V_vals, jnp.zeros((npad * 6,), f32)])

  gf = _make_sc_spmm(NF, 9, 64, 2, 64, 3, 3 * NF, True, "sc_grad_faces")(
      inp_t, G_cols, G_vals, EW, NS_w)
  lap = _make_sc_spmm(NVPAD, 7, 64, 1, 48, 1, 0, False, "sc_laplacian",
                      skew=4)(inp_t, cols_l, w_l)
  gvert = _make_sc_spmm(NVPAD, 6, 128, 1, 48, 1, 0, False, "sc_f2v",
                        skew=4)(gf, cols_v, w_v)

  a_t = coeffs.transpose(2, 1, 0).reshape(4 * CIN, COUT).T
  out = _make_mix()(inp_t, lap, gvert, a_t, bias[:, None])
  return out[None]


# speedup vs baseline: 1.0019x; 1.0019x over previous
import functools

import jax
import jax.numpy as jnp
from jax import lax
from jax.experimental import pallas as pl
from jax.experimental.pallas import tpu as pltpu
from jax.experimental.pallas import tpu_sc as plsc

NV_PREV = 10242
NV = 40962
NF = 81920
CIN = 64
COUT = 64

NC, NS = 2, 16
NW = NC * NS
NVPAD = 43008


def _split_sizes(n, parts):
  base = (n // parts) // 8 * 8
  sizes = [base] * (parts - 1)
  sizes.append(n - base * (parts - 1))
  assert all(s > 0 and s % 8 == 0 for s in sizes)
  return sizes


def _make_sc_spmm(n_rows, K, SW, wsets, R, nseg, seg_stride, fold, name,
                  skew=0):
  OW = wsets * SW
  cpw = n_rows // (NW * R)
  Kn = K // nseg
  seg_len = R * Kn
  mesh = plsc.VectorSubcoreMesh(core_axis_name="c", subcore_axis_name="s",
                                num_cores=NC, num_subcores=NS)
  gsizes = _split_sizes(R * K, 4)
  goffs = [sum(gsizes[:i]) for i in range(4)]

  if fold:
    wscratch = [pltpu.VMEM((R * K,), jnp.float32),
                pltpu.VMEM((R * K,), jnp.float32),
                pltpu.VMEM((R, 3), jnp.float32),
                pltpu.VMEM((R, 3), jnp.float32),
                pltpu.VMEM((R, 3), jnp.float32),
                pltpu.VMEM((R, 3), jnp.float32)]
  else:
    wscratch = [pltpu.VMEM((R * K,), jnp.float32),
                pltpu.VMEM((R * K,), jnp.float32)]

  @functools.partial(
      pl.kernel,
      out_type=jax.ShapeDtypeStruct((n_rows, OW), jnp.float32),
      mesh=mesh,
      scratch_types=[
          pltpu.VMEM((R * K,), jnp.int32),
          pltpu.VMEM((R * K,), jnp.int32),
          *wscratch,
          pltpu.VMEM((R * K, SW), jnp.float32),
          pltpu.VMEM((R * K, SW), jnp.float32),
          pltpu.VMEM((R, OW), jnp.float32),
          pltpu.VMEM((R, OW), jnp.float32),
          pltpu.SemaphoreType.DMA,
          pltpu.SemaphoreType.DMA,
          pltpu.SemaphoreType.DMA,
          pltpu.SemaphoreType.DMA,
      ],
      compiler_params=pltpu.CompilerParams(needs_layout_passes=False,
                                           use_tc_tiling_on_sc=False),
      name=name,
  )
  def spmm(src_hbm, cols_hbm, *rest):
    if fold:
      gv_hbm, ew_hbm, ns_hbm, out_hbm = rest[0], rest[1], rest[2], rest[3]
      (idx0, idx1, gv0, gv1, ew0, ew1, ns0, ns1, r0, r1, o0, o1,
       si0, si1, sr0, sr1) = rest[4:]
      w_b = ((gv0, ew0, ns0), (gv1, ew1, ns1))
    else:
      w_hbm, out_hbm = rest[0], rest[1]
      (idx0, idx1, w0, w1, r0, r1, o0, o1,
       si0, si1, sr0, sr1) = rest[2:]
      w_b = ((w0,), (w1,))
    idx_b, rows_b, out_b = (idx0, idx1), (r0, r1), (o0, o1)
    si, sr = (si0, si1), (sr0, sr1)
    cid = lax.axis_index("c")
    sid = lax.axis_index("s")
    if skew:
      cpw_a, cpw_b = cpw - skew, cpw + skew
      my_cpw = jnp.where(cid == 0, cpw_a, cpw_b)
      g0 = jnp.where(cid == 0, sid * cpw_a, NS * cpw_a + sid * cpw_b)
    else:
      my_cpw = cpw
      g0 = (sid * NC + cid) * cpw
    lane = lax.iota(jnp.int32, 16)
    cols_s = [lane + gi * 16 for gi in range(SW // 16)]

    def idx_copies(g, ib, wb, sem):
      ds = []
      for d in range(nseg):
        sl = pl.ds(d * seg_stride + g * seg_len, seg_len)
        ds.append(pltpu.make_async_copy(
            cols_hbm.at[sl], ib.at[pl.ds(d * seg_len, seg_len)], sem))
      if fold:
        gv_v, ew_v, ns_v = wb
        for d in range(nseg):
          sl = pl.ds(d * seg_stride + g * seg_len, seg_len)
          ds.append(pltpu.make_async_copy(
              gv_hbm.at[sl], gv_v.at[pl.ds(d * seg_len, seg_len)], sem))
        ds.append(pltpu.make_async_copy(
            ew_hbm.at[pl.ds(g * R, R)], ew_v, sem))
        ds.append(pltpu.make_async_copy(
            ns_hbm.at[pl.ds(g * R, R)], ns_v, sem))
      else:
        for d in range(nseg):
          sl = pl.ds(d * seg_stride + g * seg_len, seg_len)
          ds.append(pltpu.make_async_copy(
              w_hbm.at[sl], wb[0].at[pl.ds(d * seg_len, seg_len)], sem))
      return ds

    def gather_copies(ib, rb, sem):
      return [
          pltpu.make_async_copy(src_hbm.at[ib.at[pl.ds(o, s)]],
                                rb.at[pl.ds(o, s)], sem)
          for o, s in zip(goffs, gsizes)
      ]

    def compute(rows_v, wb, out_v):
      def f_body(f, carry):
        fvec = jnp.broadcast_to(f * Kn, (16,))
        ovec = jnp.broadcast_to(f, (16,))
        if fold:
          gv_v, ew_v, ns_v = wb
          gw = [plsc.load_gather(gv_v, [fvec + (d * seg_len + j)])
                for d in range(nseg) for j in range(Kn)]
          eww = [plsc.load_gather(ew_v, [ovec, jnp.full((16,), d, jnp.int32)])
                 for d in range(nseg)]
          nsw = [plsc.load_gather(ns_v, [ovec, jnp.full((16,), d, jnp.int32)])
                 for d in range(nseg)]
          wv = [[eww[k // Kn] * gw[k] for k in range(K)],
                [nsw[k // Kn] * gw[k] for k in range(K)]]
        else:
          wv = [[plsc.load_gather(wb[0], [fvec + (d * seg_len + j)])
                 for d in range(nseg) for j in range(Kn)]]
        rowbase = [fvec + (d * seg_len + j)
                   for d in range(nseg) for j in range(Kn)]
        for gi in range(SW // 16):
          accs = [jnp.zeros((16,), jnp.float32) for _ in range(wsets)]
          for k in range(K):
            v = plsc.load_gather(rows_v, [rowbase[k], cols_s[gi]])
            for ws in range(wsets):
              accs[ws] = accs[ws] + wv[ws][k] * v
          for ws in range(wsets):
            plsc.store_scatter(out_v, [ovec, cols_s[gi] + ws * SW], accs[ws])
        return carry

      lax.fori_loop(0, R, f_body, 0)

    for d in idx_copies(g0, idx0, w_b[0], si0):
      d.start()
    for d in idx_copies(g0, idx0, w_b[0], si0):
      d.wait()
    for d in gather_copies(idx0, r0, sr0):
      d.start()
    for d in idx_copies(g0 + 1, idx1, w_b[1], si1):
      d.start()

    def pair_body(i, carry):
      for p in range(2):
        ch = 2 * i + p
        g = g0 + ch
        q = 1 - p

        @pl.when(ch + 1 < my_cpw)
        def _fire_gather():
          for d in idx_copies(g + 1, idx_b[q], w_b[q], si[q]):
            d.wait()
          for d in gather_copies(idx_b[q], rows_b[q], sr[q]):
            d.start()

        @pl.when(ch < my_cpw)
        def _main():
          for d in gather_copies(idx_b[p], rows_b[p], sr[p]):
            d.wait()
          compute(rows_b[p], w_b[p], out_b[p])
          pltpu.sync_copy(out_b[p], out_hbm.at[pl.ds(g * R, R)])

        @pl.when(ch + 2 < my_cpw)
        def _prefetch_idx():
          for d in idx_copies(g + 2, idx_b[p], w_b[p], si[p]):
            d.start()

      return carry

    lax.fori_loop(0, (cpw + abs(skew)) // 2, pair_body, 0)

  return spmm


_NBLK = 512
_NGRID = (NV + _NBLK - 1) // _NBLK


def _mix_kernel(inp_ref, lap_ref, gv_ref, at_ref, bias_ref, out_ref):
  feat = jnp.concatenate([inp_ref[...], lap_ref[...], gv_ref[...]], axis=1)
  out = lax.dot_general(at_ref[...], feat, (((1,), (1,)), ((), ())),
                        preferred_element_type=jnp.float32)
  out_ref[...] = out + bias_ref[...]


@functools.lru_cache(maxsize=None)
def _make_mix():
  return pl.pallas_call(
      _mix_kernel,
      grid=(_NGRID,),
      in_specs=[
          pl.BlockSpec((_NBLK, CIN), lambda i: (i, 0)),
          pl.BlockSpec((_NBLK, CIN), lambda i: (i, 0)),
          pl.BlockSpec((_NBLK, 2 * CIN), lambda i: (i, 0)),
          pl.BlockSpec((COUT, 4 * CIN), lambda i: (0, 0)),
          pl.BlockSpec((COUT, 1), lambda i: (0, 0)),
      ],
      out_specs=pl.BlockSpec((COUT, _NBLK), lambda i: (0, i)),
      out_shape=jax.ShapeDtypeStruct((COUT, NV), jnp.float32),
  )


def kernel(x, verts, G_rows, G_cols, G_vals, NS_w, EW, L_rows, L_cols, L_vals,
           F2V_rows, F2V_cols, F2V_vals, coeffs, bias):
  f32 = jnp.float32
  i32 = jnp.int32
  inp_t = jnp.concatenate(
      [x[0].T, jnp.ones((NVPAD - NV_PREV, CIN), f32)], axis=0)

  npad = NVPAD - NV
  cols_l = jnp.concatenate([L_cols, jnp.zeros((npad * 7,), i32)])
  w_l = jnp.concatenate([L_vals, jnp.zeros((npad * 7,), f32)])
  cols_v = jnp.concatenate([F2V_cols, jnp.zeros((npad * 6,), i32)])
  w_v = jnp.concatenate([F2V_vals, jnp.zeros((npad * 6,), f32)])

  gf = _make_sc_spmm(NF, 9, 64, 2, 64, 3, 3 * NF, True, "sc_grad_faces")(
      inp_t, G_cols, G_vals, EW, NS_w)
  lap = _make_sc_spmm(NVPAD, 7, 64, 1, 48, 1, 0, False, "sc_laplacian",
                      skew=-4)(inp_t, cols_l, w_l)
  gvert = _make_sc_spmm(NVPAD, 6, 128, 1, 48, 1, 0, False, "sc_f2v",
                        skew=-4)(gf, cols_v, w_v)

  a_t = coeffs.transpose(2, 1, 0).reshape(4 * CIN, COUT).T
  out = _make_mix()(inp_t, lap, gvert, a_t, bias[:, None])
  return out[None]


# speedup vs baseline: 1.1131x; 1.1110x over previous
import functools

import jax
import jax.numpy as jnp
from jax import lax
from jax.experimental import pallas as pl
from jax.experimental.pallas import tpu as pltpu
from jax.experimental.pallas import tpu_sc as plsc

NV_PREV = 10242
NV = 40962
NF = 81920
CIN = 64
COUT = 64

NC, NS = 2, 16
NW = NC * NS
NVPAD = 43008


def _split_sizes(n, parts):
  base = (n // parts) // 8 * 8
  sizes = [base] * (parts - 1)
  sizes.append(n - base * (parts - 1))
  assert all(s > 0 and s % 8 == 0 for s in sizes)
  return sizes


def _make_sc_spmm(n_rows, K, SW, wsets, R, nseg, seg_stride, fold, name,
                  skew=0):
  OW = wsets * SW
  cpw = n_rows // (NW * R)
  Kn = K // nseg
  seg_len = R * Kn
  mesh = plsc.VectorSubcoreMesh(core_axis_name="c", subcore_axis_name="s",
                                num_cores=NC, num_subcores=NS)
  gsizes = _split_sizes(R * K, 4)
  goffs = [sum(gsizes[:i]) for i in range(4)]

  if fold:
    wscratch = [pltpu.VMEM((R * K,), jnp.float32),
                pltpu.VMEM((R * K,), jnp.float32),
                pltpu.VMEM((R, 3), jnp.float32),
                pltpu.VMEM((R, 3), jnp.float32),
                pltpu.VMEM((R, 3), jnp.float32),
                pltpu.VMEM((R, 3), jnp.float32)]
  else:
    wscratch = [pltpu.VMEM((R * K,), jnp.float32),
                pltpu.VMEM((R * K,), jnp.float32)]

  @functools.partial(
      pl.kernel,
      out_type=jax.ShapeDtypeStruct((n_rows, OW), jnp.float32),
      mesh=mesh,
      scratch_types=[
          pltpu.VMEM((R * K,), jnp.int32),
          pltpu.VMEM((R * K,), jnp.int32),
          *wscratch,
          pltpu.VMEM((R * K, SW), jnp.float32),
          pltpu.VMEM((R * K, SW), jnp.float32),
          pltpu.VMEM((R, OW), jnp.float32),
          pltpu.VMEM((R, OW), jnp.float32),
          pltpu.SemaphoreType.DMA,
          pltpu.SemaphoreType.DMA,
          pltpu.SemaphoreType.DMA,
          pltpu.SemaphoreType.DMA,
      ],
      compiler_params=pltpu.CompilerParams(needs_layout_passes=False,
                                           use_tc_tiling_on_sc=False),
      name=name,
  )
  def spmm(src_hbm, cols_hbm, *rest):
    if fold:
      gv_hbm, ew_hbm, ns_hbm, out_hbm = rest[0], rest[1], rest[2], rest[3]
      (idx0, idx1, gv0, gv1, ew0, ew1, ns0, ns1, r0, r1, o0, o1,
       si0, si1, sr0, sr1) = rest[4:]
      w_b = ((gv0, ew0, ns0), (gv1, ew1, ns1))
    else:
      w_hbm, out_hbm = rest[0], rest[1]
      (idx0, idx1, w0, w1, r0, r1, o0, o1,
       si0, si1, sr0, sr1) = rest[2:]
      w_b = ((w0,), (w1,))
    idx_b, rows_b, out_b = (idx0, idx1), (r0, r1), (o0, o1)
    si, sr = (si0, si1), (sr0, sr1)
    cid = lax.axis_index("c")
    sid = lax.axis_index("s")
    if skew:
      cpw_a, cpw_b = cpw - skew, cpw + skew
      my_cpw = jnp.where(cid == 0, cpw_a, cpw_b)
      g0 = jnp.where(cid == 0, sid * cpw_a, NS * cpw_a + sid * cpw_b)
    else:
      my_cpw = cpw
      g0 = (sid * NC + cid) * cpw
    lane = lax.iota(jnp.int32, 16)
    cols_s = [lane + gi * 16 for gi in range(SW // 16)]

    def idx_copies(g, ib, wb, sem):
      ds = []
      for d in range(nseg):
        sl = pl.ds(d * seg_stride + g * seg_len, seg_len)
        ds.append(pltpu.make_async_copy(
            cols_hbm.at[sl], ib.at[pl.ds(d * seg_len, seg_len)], sem))
      if fold:
        gv_v, ew_v, ns_v = wb
        for d in range(nseg):
          sl = pl.ds(d * seg_stride + g * seg_len, seg_len)
          ds.append(pltpu.make_async_copy(
              gv_hbm.at[sl], gv_v.at[pl.ds(d * seg_len, seg_len)], sem))
        ds.append(pltpu.make_async_copy(
            ew_hbm.at[pl.ds(g * R, R)], ew_v, sem))
        ds.append(pltpu.make_async_copy(
            ns_hbm.at[pl.ds(g * R, R)], ns_v, sem))
      else:
        for d in range(nseg):
          sl = pl.ds(d * seg_stride + g * seg_len, seg_len)
          ds.append(pltpu.make_async_copy(
              w_hbm.at[sl], wb[0].at[pl.ds(d * seg_len, seg_len)], sem))
      return ds

    def gather_copies(ib, rb, sem):
      return [
          pltpu.make_async_copy(src_hbm.at[ib.at[pl.ds(o, s)]],
                                rb.at[pl.ds(o, s)], sem)
          for o, s in zip(goffs, gsizes)
      ]

    def compute(rows_v, wb, out_v):
      def f_body(f, carry):
        fvec = jnp.broadcast_to(f * Kn, (16,))
        ovec = jnp.broadcast_to(f, (16,))
        if fold:
          gv_v, ew_v, ns_v = wb
          gw = [plsc.load_gather(gv_v, [fvec + (d * seg_len + j)])
                for d in range(nseg) for j in range(Kn)]
          eww = [plsc.load_gather(ew_v, [ovec, jnp.full((16,), d, jnp.int32)])
                 for d in range(nseg)]
          nsw = [plsc.load_gather(ns_v, [ovec, jnp.full((16,), d, jnp.int32)])
                 for d in range(nseg)]
          wv = [[eww[k // Kn] * gw[k] for k in range(K)],
                [nsw[k // Kn] * gw[k] for k in range(K)]]
        else:
          wv = [[plsc.load_gather(wb[0], [fvec + (d * seg_len + j)])
                 for d in range(nseg) for j in range(Kn)]]
        rowbase = [fvec + (d * seg_len + j)
                   for d in range(nseg) for j in range(Kn)]
        for gi in range(SW // 16):
          accs = [jnp.zeros((16,), jnp.float32) for _ in range(wsets)]
          for k in range(K):
            v = plsc.load_gather(rows_v, [rowbase[k], cols_s[gi]])
            for ws in range(wsets):
              accs[ws] = accs[ws] + wv[ws][k] * v
          for ws in range(wsets):
            plsc.store_scatter(out_v, [ovec, cols_s[gi] + ws * SW], accs[ws])
        return carry

      lax.fori_loop(0, R, f_body, 0)

    for d in idx_copies(g0, idx0, w_b[0], si0):
      d.start()
    for d in idx_copies(g0, idx0, w_b[0], si0):
      d.wait()
    for d in gather_copies(idx0, r0, sr0):
      d.start()
    for d in idx_copies(g0 + 1, idx1, w_b[1], si1):
      d.start()

    def pair_body(i, carry):
      for p in range(2):
        ch = 2 * i + p
        g = g0 + ch
        q = 1 - p

        @pl.when(ch + 1 < my_cpw)
        def _fire_gather():
          for d in idx_copies(g + 1, idx_b[q], w_b[q], si[q]):
            d.wait()
          for d in gather_copies(idx_b[q], rows_b[q], sr[q]):
            d.start()

        @pl.when(ch < my_cpw)
        def _main():
          for d in gather_copies(idx_b[p], rows_b[p], sr[p]):
            d.wait()
          compute(rows_b[p], w_b[p], out_b[p])
          pltpu.sync_copy(out_b[p], out_hbm.at[pl.ds(g * R, R)])

        @pl.when(ch + 2 < my_cpw)
        def _prefetch_idx():
          for d in idx_copies(g + 2, idx_b[p], w_b[p], si[p]):
            d.start()

      return carry

    lax.fori_loop(0, (cpw + abs(skew)) // 2, pair_body, 0)

  return spmm


_NBLK = 512
_NGRID = (NV + _NBLK - 1) // _NBLK


def _mix_kernel(inp_ref, lap_ref, gv_ref, at_ref, bias_ref, out_ref):
  feat = jnp.concatenate([inp_ref[...], lap_ref[...], gv_ref[...]], axis=1)
  out = lax.dot_general(at_ref[...], feat, (((1,), (1,)), ((), ())),
                        preferred_element_type=jnp.float32)
  out_ref[...] = out + bias_ref[...]


@functools.lru_cache(maxsize=None)
def _make_mix():
  return pl.pallas_call(
      _mix_kernel,
      grid=(_NGRID,),
      in_specs=[
          pl.BlockSpec((_NBLK, CIN), lambda i: (i, 0)),
          pl.BlockSpec((_NBLK, CIN), lambda i: (i, 0)),
          pl.BlockSpec((_NBLK, 2 * CIN), lambda i: (i, 0)),
          pl.BlockSpec((COUT, 4 * CIN), lambda i: (0, 0)),
          pl.BlockSpec((COUT, 1), lambda i: (0, 0)),
      ],
      out_specs=pl.BlockSpec((COUT, _NBLK), lambda i: (0, i)),
      out_shape=jax.ShapeDtypeStruct((COUT, NV), jnp.float32),
  )


def kernel(x, verts, G_rows, G_cols, G_vals, NS_w, EW, L_rows, L_cols, L_vals,
           F2V_rows, F2V_cols, F2V_vals, coeffs, bias):
  f32 = jnp.float32
  i32 = jnp.int32
  inp_t = jnp.concatenate(
      [x[0].T, jnp.ones((NVPAD - NV_PREV, CIN), f32)], axis=0)

  npad = NVPAD - NV
  cols_l = jnp.concatenate([L_cols, jnp.zeros((npad * 7,), i32)])
  w_l = jnp.concatenate([L_vals, jnp.zeros((npad * 7,), f32)])
  cols_v = jnp.concatenate([F2V_cols, jnp.zeros((npad * 6,), i32)])
  w_v = jnp.concatenate([F2V_vals, jnp.zeros((npad * 6,), f32)])

  gf = _make_sc_spmm(NF, 9, 64, 2, 64, 3, 3 * NF, True, "sc_grad_faces")(
      inp_t, G_cols, G_vals, EW, NS_w)
  lap = _make_sc_spmm(NVPAD, 7, 64, 1, 48, 1, 0, False, "sc_laplacian",
                      skew=0)(inp_t, cols_l, w_l)
  gvert = _make_sc_spmm(NVPAD, 6, 128, 1, 48, 1, 0, False, "sc_f2v",
                        skew=0)(gf, cols_v, w_v)

  a_t = coeffs.transpose(2, 1, 0).reshape(4 * CIN, COUT).T
  out = _make_mix()(inp_t, lap, gvert, a_t, bias[:, None])
  return out[None]
